# Initial kernel scaffold; baseline (speedup 1.0000x reference)
import jax
import jax.numpy as jnp
from jax.experimental import pallas as pl


def kernel(indices, table):
    raise NotImplementedError("write your pallas kernel here")


# speedup vs baseline: 1.4352x; 1.4352x over previous
import functools

import jax
import jax.numpy as jnp
from jax import lax
from jax.experimental import pallas as pl
from jax.experimental.pallas import tpu as pltpu
from jax.experimental.pallas import tpu_sc as plsc

_NUM_WORKERS = 32
_ROWS_PER_DMA = 128
_DIM = 32


@functools.lru_cache(maxsize=None)
def _make_gather(n_total: int, dim: int):
    assert n_total % (_NUM_WORKERS * _ROWS_PER_DMA) == 0
    per_worker = n_total // _NUM_WORKERS
    steps = per_worker // _ROWS_PER_DMA

    mesh = plsc.VectorSubcoreMesh(core_axis_name="c", subcore_axis_name="s")

    @functools.partial(
        pl.kernel,
        mesh=mesh,
        out_type=jax.ShapeDtypeStruct((n_total, dim), jnp.float32),
        scratch_types=[
            pltpu.VMEM((steps, _ROWS_PER_DMA), jnp.int32),
            pltpu.VMEM((_ROWS_PER_DMA, dim), jnp.float32),
            pltpu.SemaphoreType.DMA,
        ],
        compiler_params=pltpu.CompilerParams(use_tc_tiling_on_sc=False),
    )
    def gather_kernel(idx_hbm, table_hbm, out_hbm, idx_v, rows_v, sem):
        wid = lax.axis_index("s") * 2 + lax.axis_index("c")
        base = wid * per_worker
        pltpu.sync_copy(idx_hbm.at[wid], idx_v)

        def step_fn(i, carry):
            pltpu.async_copy(table_hbm.at[idx_v.at[i]], rows_v, sem).wait()
            pltpu.sync_copy(
                rows_v, out_hbm.at[pl.ds(base + i * _ROWS_PER_DMA, _ROWS_PER_DMA)]
            )
            return carry

        lax.fori_loop(0, steps, step_fn, 0)

    return gather_kernel


def kernel(indices, table):
    shape = indices.shape
    dim = table.shape[1]
    flat = indices.reshape(-1).astype(jnp.int32)
    n = flat.shape[0]
    group = _NUM_WORKERS * _ROWS_PER_DMA
    n_pad = ((n + group - 1) // group) * group
    if n_pad != n:
        flat = jnp.concatenate([flat, jnp.zeros((n_pad - n,), jnp.int32)])
    idx3 = flat.reshape(_NUM_WORKERS, n_pad // (_NUM_WORKERS * _ROWS_PER_DMA),
                        _ROWS_PER_DMA)
    out = _make_gather(n_pad, dim)(idx3, table)
    if n_pad != n:
        out = out[:n]
    return out.reshape(*shape, dim)


# speedup vs baseline: 1.5765x; 1.0985x over previous
import functools

import jax
import jax.numpy as jnp
from jax import lax
from jax.experimental import pallas as pl
from jax.experimental.pallas import tpu as pltpu
from jax.experimental.pallas import tpu_sc as plsc

_NUM_WORKERS = 32
_ROWS_PER_DMA = 128
_SUBS = 8
_CHUNK = _SUBS * _ROWS_PER_DMA


@functools.lru_cache(maxsize=None)
def _make_gather(n_total: int, dim: int):
    assert n_total % (_NUM_WORKERS * _CHUNK) == 0
    per_worker = n_total // _NUM_WORKERS
    nchunks = per_worker // _CHUNK
    nsteps = per_worker // _ROWS_PER_DMA

    mesh = plsc.VectorSubcoreMesh(core_axis_name="c", subcore_axis_name="s")

    @functools.partial(
        pl.kernel,
        mesh=mesh,
        out_type=jax.ShapeDtypeStruct((n_total, dim), jnp.float32),
        scratch_types=[
            pltpu.VMEM((nsteps, _ROWS_PER_DMA), jnp.int32),
            pltpu.VMEM((2, _CHUNK, dim), jnp.float32),
            pltpu.SemaphoreType.DMA,
        ],
        compiler_params=pltpu.CompilerParams(use_tc_tiling_on_sc=False),
    )
    def gather_kernel(idx_hbm, table_hbm, out_hbm, idx_v, rows_v, sem):
        wid = lax.axis_index("s") * 2 + lax.axis_index("c")
        base = wid * per_worker
        pltpu.sync_copy(idx_hbm.at[wid], idx_v)

        def fire(g, b):
            for j in range(_SUBS):
                pltpu.async_copy(
                    table_hbm.at[idx_v.at[g * _SUBS + j]],
                    rows_v.at[b, pl.ds(j * _ROWS_PER_DMA, _ROWS_PER_DMA)],
                    sem,
                )

        def drain_and_scatter(g, b):
            for j in range(_SUBS):
                pltpu.make_async_copy(
                    table_hbm.at[idx_v.at[g * _SUBS + j]],
                    rows_v.at[b, pl.ds(j * _ROWS_PER_DMA, _ROWS_PER_DMA)],
                    sem,
                ).wait()
            pltpu.sync_copy(
                rows_v.at[b], out_hbm.at[pl.ds(base + g * _CHUNK, _CHUNK)]
            )

        fire(0, 0)

        def body(g, carry):
            fire(g + 1, lax.rem(g + 1, 2))
            drain_and_scatter(g, lax.rem(g, 2))
            return carry

        lax.fori_loop(0, nchunks - 1, body, 0)
        drain_and_scatter(nchunks - 1, (nchunks - 1) % 2)

    return gather_kernel


def kernel(indices, table):
    shape = indices.shape
    dim = table.shape[1]
    flat = indices.reshape(-1).astype(jnp.int32)
    n = flat.shape[0]
    group = _NUM_WORKERS * _CHUNK
    n_pad = ((n + group - 1) // group) * group
    if n_pad != n:
        flat = jnp.concatenate([flat, jnp.zeros((n_pad - n,), jnp.int32)])
    idx3 = flat.reshape(_NUM_WORKERS, -1, _ROWS_PER_DMA)
    out = _make_gather(n_pad, dim)(idx3, table)
    if n_pad != n:
        out = out[:n]
    return out.reshape(*shape, dim)


# speedup vs baseline: 1.7817x; 1.1301x over previous
import functools

import jax
import jax.numpy as jnp
from jax import lax
from jax.experimental import pallas as pl
from jax.experimental.pallas import tpu as pltpu
from jax.experimental.pallas import tpu_sc as plsc

_NUM_WORKERS = 32
_ROWS_PER_DMA = 128
_SUBS = 8
_CHUNK = _SUBS * _ROWS_PER_DMA


@functools.lru_cache(maxsize=None)
def _make_gather(n_total: int, dim: int):
    assert n_total % (_NUM_WORKERS * _CHUNK) == 0
    per_worker = n_total // _NUM_WORKERS
    nchunks = per_worker // _CHUNK
    nsteps = per_worker // _ROWS_PER_DMA

    mesh = plsc.VectorSubcoreMesh(core_axis_name="c", subcore_axis_name="s")

    @functools.partial(
        pl.kernel,
        mesh=mesh,
        out_type=jax.ShapeDtypeStruct((n_total, dim), jnp.float32),
        scratch_types=[
            pltpu.VMEM((nsteps, _ROWS_PER_DMA), jnp.int32),
            pltpu.VMEM((2, _CHUNK, dim), jnp.float32),
            pltpu.SemaphoreType.DMA,
        ],
        compiler_params=pltpu.CompilerParams(use_tc_tiling_on_sc=False),
    )
    def gather_kernel(idx_hbm, table_hbm, out_hbm, idx_v, rows_v, sem):
        wid = lax.axis_index("s") * 2 + lax.axis_index("c")
        base = wid * per_worker
        pltpu.sync_copy(idx_hbm.at[wid], idx_v)

        def fire(g, b):
            for j in range(_SUBS):
                pltpu.async_copy(
                    table_hbm.at[idx_v.at[g * _SUBS + j]],
                    rows_v.at[b, pl.ds(j * _ROWS_PER_DMA, _ROWS_PER_DMA)],
                    sem,
                )

        def drain_and_scatter(g, b):
            for j in range(_SUBS):
                pltpu.make_async_copy(
                    table_hbm.at[idx_v.at[g * _SUBS + j]],
                    rows_v.at[b, pl.ds(j * _ROWS_PER_DMA, _ROWS_PER_DMA)],
                    sem,
                ).wait()
            pltpu.sync_copy(
                rows_v.at[b], out_hbm.at[pl.ds(base + g * _CHUNK, _CHUNK)]
            )

        fire(0, 0)

        def body(g, carry):
            fire(g + 1, lax.rem(g + 1, 2))
            drain_and_scatter(g, lax.rem(g, 2))
            return carry

        lax.fori_loop(0, nchunks - 1, body, 0)
        drain_and_scatter(nchunks - 1, (nchunks - 1) % 2)

    return gather_kernel


_TR_BLK = 8192


@functools.lru_cache(maxsize=None)
def _make_transpose(v: int, dim: int):
    nblk = (v + _TR_BLK - 1) // _TR_BLK

    fold = 128 // dim

    def body(x_ref, o_ref):
        x3 = x_ref[...].T.reshape(_TR_BLK // fold, fold, dim)
        o_ref[...] = jnp.concatenate(
            [x3[:, k, :] for k in range(fold)], axis=1
        ).reshape(-1)

    return pl.pallas_call(
        body,
        grid=(nblk,),
        in_specs=[pl.BlockSpec((dim, _TR_BLK), lambda i: (0, i))],
        out_specs=pl.BlockSpec((_TR_BLK * dim,), lambda i: (i,)),
        out_shape=jax.ShapeDtypeStruct((v * dim,), jnp.float32),
    )


def kernel(indices, table):
    shape = indices.shape
    v, dim = table.shape
    table_rm = _make_transpose(v, dim)(table.T).reshape(v, dim)
    flat = indices.reshape(-1).astype(jnp.int32)
    n = flat.shape[0]
    group = _NUM_WORKERS * _CHUNK
    n_pad = ((n + group - 1) // group) * group
    if n_pad != n:
        flat = jnp.concatenate([flat, jnp.zeros((n_pad - n,), jnp.int32)])
    idx3 = flat.reshape(_NUM_WORKERS, -1, _ROWS_PER_DMA)
    out = _make_gather(n_pad, dim)(idx3, table_rm)
    if n_pad != n:
        out = out[:n]
    return out.reshape(*shape, dim)
